# baseline (speedup 1.0000x reference)
import functools

import jax
import jax.numpy as jnp
from jax import lax
from jax.experimental import pallas as pl
from jax.experimental.pallas import tpu as pltpu
from jax.experimental.pallas import tpu_sc as plsc

NUM_CLASSES = 1000
HALF = 64
OUT_DIM = 2 * HALF
B, N = 4096, 200
TOT = B * N
TOT2 = TOT // 2

NC, NS = 2, 16
NW = NC * NS
PER_W2 = TOT2 // NW
CH = 128
KP = 2
GRP = KP * CH
N_GRP = PER_W2 // GRP


def _sc_gather(ids_even, ids_odd, table):
    mesh = plsc.VectorSubcoreMesh(core_axis_name="c", subcore_axis_name="s")

    @functools.partial(
        pl.kernel,
        out_type=jax.ShapeDtypeStruct((TOT2, OUT_DIM), jnp.float32),
        mesh=mesh,
        compiler_params=pltpu.CompilerParams(use_tc_tiling_on_sc=False),
        scratch_types=[
            pltpu.VMEM((PER_W2,), jnp.int32),
            pltpu.VMEM((PER_W2,), jnp.int32),
            pltpu.VMEM((GRP, HALF), jnp.float32),
            pltpu.VMEM((GRP, HALF), jnp.float32),
            pltpu.VMEM((GRP, HALF), jnp.float32),
            pltpu.VMEM((GRP, HALF), jnp.float32),
            pltpu.VMEM_SHARED((NUM_CLASSES, HALF), jnp.float32),
            pltpu.SemaphoreType.DMA,
            pltpu.SemaphoreType.DMA,
            pltpu.SemaphoreType.DMA,
            pltpu.SemaphoreType.DMA,
        ],
    )
    def sc_body(ide_hbm, ido_hbm, table_hbm, out_hbm, idx_e, idx_o,
                re0, ro0, re1, ro1, tab_s, g0, g1, w0, w1):
        cid = lax.axis_index("c")
        sid = lax.axis_index("s")
        wid = sid * NC + cid
        base = wid * PER_W2

        @pl.when(sid == 0)
        def _stage_table():
            pltpu.sync_copy(table_hbm, tab_s)

        plsc.subcore_barrier()
        pltpu.sync_copy(ide_hbm.at[pl.ds(base, PER_W2)], idx_e)
        pltpu.sync_copy(ido_hbm.at[pl.ds(base, PER_W2)], idx_o)

        def issue_gathers(g, re, ro, gsem):
            for j in range(KP):
                r0 = j * CH
                i0 = g * GRP + j * CH
                pltpu.async_copy(
                    tab_s.at[idx_e.at[pl.ds(i0, CH)]],
                    re.at[pl.ds(r0, CH)], gsem,
                )
                pltpu.async_copy(
                    tab_s.at[idx_o.at[pl.ds(i0, CH)]],
                    ro.at[pl.ds(r0, CH)], gsem,
                )

        def drain_gathers(re, ro, gsem):
            pltpu.make_async_copy(
                out_hbm.at[pl.ds(0, GRP), pl.ds(0, HALF)], re, gsem).wait()
            pltpu.make_async_copy(
                out_hbm.at[pl.ds(0, GRP), pl.ds(0, HALF)], ro, gsem).wait()

        def issue_write(g, re, ro, wsem):
            r0 = base + g * GRP
            pltpu.async_copy(
                re, out_hbm.at[pl.ds(r0, GRP), pl.ds(0, HALF)], wsem)
            pltpu.async_copy(
                ro, out_hbm.at[pl.ds(r0, GRP), pl.ds(HALF, HALF)], wsem)

        def drain_write(re, ro, wsem):
            pltpu.make_async_copy(
                re, out_hbm.at[pl.ds(0, GRP), pl.ds(0, HALF)], wsem).wait()
            pltpu.make_async_copy(
                ro, out_hbm.at[pl.ds(0, GRP), pl.ds(0, HALF)], wsem).wait()

        issue_gathers(0, re0, ro0, g0)
        issue_gathers(1, re1, ro1, g1)

        @pl.loop(0, N_GRP, step=2)
        def _grp(g):
            drain_gathers(re0, ro0, g0)
            issue_write(g, re0, ro0, w0)
            drain_gathers(re1, ro1, g1)
            issue_write(g + 1, re1, ro1, w1)

            @pl.when(g + 2 < N_GRP)
            def _refill0():
                drain_write(re0, ro0, w0)
                issue_gathers(g + 2, re0, ro0, g0)

            @pl.when(g + 3 < N_GRP)
            def _refill1():
                drain_write(re1, ro1, w1)
                issue_gathers(g + 3, re1, ro1, g1)

        drain_write(re0, ro0, w0)
        drain_write(re1, ro1, w1)

    return sc_body(ids_even, ids_odd, table)


BB = 128
TB = BB * N
TB2 = TB // 2
N2 = N // 2


def _tc_body(gath_ref, c4_ref, w1_ref, b1_ref, w2_ref, b2_ref, out_ref):
    c4 = c4_ref[...]

    def mlp(lhs):
        h = (
            jax.lax.dot_general(
                lhs, w1_ref[...], (((0,), (0,)), ((), ())),
                preferred_element_type=jnp.float32,
            )
            + b1_ref[...]
        )
        h = jnp.maximum(h, 0.0)
        return (
            jax.lax.dot_general(
                h, w2_ref[...], (((1,), (0,)), ((), ())),
                preferred_element_type=jnp.float32,
            )
            + b2_ref[...]
        )

    ce_e = mlp(jnp.transpose(c4[:, :, 0, :], (0, 2, 1)).reshape(3, TB2))
    ce_o = mlp(jnp.transpose(c4[:, :, 1, :], (0, 2, 1)).reshape(3, TB2))
    g = gath_ref[...]
    out_ref[...] = jnp.concatenate(
        [g[:, :HALF], ce_e, g[:, HALF:], ce_o], axis=1
    )


def _tc_mlp(sc_out, coords_t4, W1, b1, W2, b2):
    grid = (B // BB,)
    return pl.pallas_call(
        _tc_body,
        grid=grid,
        in_specs=[
            pl.BlockSpec((TB2, OUT_DIM), lambda i: (i, 0)),
            pl.BlockSpec((3, N2, 2, BB), lambda i: (0, 0, 0, i)),
            pl.BlockSpec((3, HALF), lambda i: (0, 0)),
            pl.BlockSpec((1, HALF), lambda i: (0, 0)),
            pl.BlockSpec((HALF, HALF), lambda i: (0, 0)),
            pl.BlockSpec((1, HALF), lambda i: (0, 0)),
        ],
        out_specs=pl.BlockSpec((TB2, 2 * OUT_DIM), lambda i: (i, 0)),
        out_shape=jax.ShapeDtypeStruct((TOT2, 2 * OUT_DIM), jnp.float32),
        compiler_params=pltpu.CompilerParams(vmem_limit_bytes=100 * 1024 * 1024),
    )(sc_out, coords_t4, W1, b1, W2, b2)


def kernel(class_ids, coords, emb_table, W1, b1, W2, b2):
    ids2 = class_ids.reshape(TOT2, 2).astype(jnp.int32)
    ids_even = ids2[:, 0]
    ids_odd = ids2[:, 1]
    coords_t4 = jnp.transpose(coords, (2, 1, 0)).reshape(3, N2, 2, B)
    sc_out = _sc_gather(ids_even, ids_odd, emb_table)
    out = _tc_mlp(
        sc_out, coords_t4, W1, b1.reshape(1, HALF), W2, b2.reshape(1, HALF)
    )
    return out.reshape(B, N, OUT_DIM)


# speedup vs baseline: 2.5612x; 2.5612x over previous
import functools

import jax
import jax.numpy as jnp
from jax import lax
from jax.experimental import pallas as pl
from jax.experimental.pallas import tpu as pltpu
from jax.experimental.pallas import tpu_sc as plsc

NUM_CLASSES = 1000
HALF = 64
OUT_DIM = 2 * HALF
B, N = 4096, 200
TOT = B * N

NC, NS = 2, 16
NW = NC * NS
PER_W = TOT // NW
CH = 128
K = 4
GRP = K * CH
N_GRP = PER_W // GRP


def _sc_gather(ids_flat, table):
    mesh = plsc.VectorSubcoreMesh(core_axis_name="c", subcore_axis_name="s")

    @functools.partial(
        pl.kernel,
        out_type=jax.ShapeDtypeStruct((TOT, OUT_DIM), jnp.float32),
        mesh=mesh,
        compiler_params=pltpu.CompilerParams(use_tc_tiling_on_sc=False),
        scratch_types=[
            pltpu.VMEM((PER_W,), jnp.int32),
            pltpu.VMEM((GRP, HALF), jnp.float32),
            pltpu.VMEM((GRP, HALF), jnp.float32),
            pltpu.VMEM_SHARED((NUM_CLASSES, HALF), jnp.float32),
            pltpu.SemaphoreType.DMA,
            pltpu.SemaphoreType.DMA,
            pltpu.SemaphoreType.DMA,
            pltpu.SemaphoreType.DMA,
        ],
    )
    def sc_body(ids_hbm, table_hbm, out_hbm, idx_v, rows0, rows1, tab_s,
                g0, g1, w0, w1):
        cid = lax.axis_index("c")
        sid = lax.axis_index("s")
        wid = sid * NC + cid
        base = wid * PER_W

        @pl.when(sid == 0)
        def _stage_table():
            pltpu.sync_copy(table_hbm, tab_s)

        plsc.subcore_barrier()
        pltpu.sync_copy(ids_hbm.at[pl.ds(base, PER_W)], idx_v)

        def issue_gathers(g, rows, gsem):
            for j in range(K):
                pltpu.async_copy(
                    tab_s.at[idx_v.at[pl.ds(g * GRP + j * CH, CH)]],
                    rows.at[pl.ds(j * CH, CH)],
                    gsem,
                )

        def drain_gathers(rows, gsem):
            pltpu.make_async_copy(tab_s.at[pl.ds(0, GRP)], rows, gsem).wait()

        def issue_write(g, rows, wsem):
            pltpu.async_copy(
                rows,
                out_hbm.at[pl.ds(base + g * GRP, GRP), pl.ds(0, HALF)],
                wsem,
            )

        def drain_write(rows, wsem):
            pltpu.make_async_copy(
                rows, out_hbm.at[pl.ds(0, GRP), pl.ds(0, HALF)], wsem).wait()

        issue_gathers(0, rows0, g0)
        issue_gathers(1, rows1, g1)

        @pl.loop(0, N_GRP, step=2)
        def _grp(g):
            drain_gathers(rows0, g0)
            issue_write(g, rows0, w0)
            drain_gathers(rows1, g1)
            issue_write(g + 1, rows1, w1)

            @pl.when(g + 2 < N_GRP)
            def _refill0():
                drain_write(rows0, w0)
                issue_gathers(g + 2, rows0, g0)

            @pl.when(g + 3 < N_GRP)
            def _refill1():
                drain_write(rows1, w1)
                issue_gathers(g + 3, rows1, g1)

        drain_write(rows0, w0)
        drain_write(rows1, w1)

    return sc_body(ids_flat, table)


BB = 128
TB = BB * N


def _tc_body(buf_ref, c3_ref, w1_ref, b1_ref, w2_ref, b2_ref, out_ref):
    c = c3_ref[...]
    ct = jnp.transpose(c, (0, 2, 1))
    lhs = ct.reshape(3, TB)
    h = (
        jax.lax.dot_general(
            lhs, w1_ref[...], (((0,), (0,)), ((), ())),
            preferred_element_type=jnp.float32,
        )
        + b1_ref[...]
    )
    h = jnp.maximum(h, 0.0)
    coord_emb = (
        jax.lax.dot_general(
            h, w2_ref[...], (((1,), (0,)), ((), ())),
            preferred_element_type=jnp.float32,
        )
        + b2_ref[...]
    )
    out_ref[...] = jnp.concatenate([buf_ref[:, :HALF], coord_emb], axis=1)


def _tc_mlp(sc_out, coords_t, W1, b1, W2, b2):
    grid = (B // BB,)
    return pl.pallas_call(
        _tc_body,
        grid=grid,
        in_specs=[
            pl.BlockSpec((TB, OUT_DIM), lambda i: (i, 0)),
            pl.BlockSpec((3, N, BB), lambda i: (0, 0, i)),
            pl.BlockSpec((3, HALF), lambda i: (0, 0)),
            pl.BlockSpec((1, HALF), lambda i: (0, 0)),
            pl.BlockSpec((HALF, HALF), lambda i: (0, 0)),
            pl.BlockSpec((1, HALF), lambda i: (0, 0)),
        ],
        out_specs=pl.BlockSpec((TB, OUT_DIM), lambda i: (i, 0)),
        out_shape=jax.ShapeDtypeStruct((TOT, OUT_DIM), jnp.float32),
        input_output_aliases={0: 0},
        compiler_params=pltpu.CompilerParams(vmem_limit_bytes=100 * 1024 * 1024),
    )(sc_out, coords_t, W1, b1, W2, b2)


def kernel(class_ids, coords, emb_table, W1, b1, W2, b2):
    ids_flat = class_ids.reshape(TOT).astype(jnp.int32)
    coords_t = jnp.transpose(coords, (2, 1, 0))
    sc_out = _sc_gather(ids_flat, emb_table)
    out = _tc_mlp(
        sc_out, coords_t, W1, b1.reshape(1, HALF), W2, b2.reshape(1, HALF)
    )
    return out.reshape(B, N, OUT_DIM)
